# baseline (device time: 169348 ns/iter reference)
import jax
import jax.numpy as jnp
from jax import lax
from jax.experimental import pallas as pl
from jax.experimental.pallas import tpu as pltpu

N_DEV = 32
LOG2 = 5
N_LAYERS = 3


def kernel(x, Win0, Wout0, Win1, Wout1, Win2, Wout2):
    b, d = x.shape

    def body(x_ref, win0_ref, wout0_ref, win1_ref, wout1_ref, win2_ref,
             wout2_ref, out_ref, accum_ref, recv_ref, send_sems, recv_sems):
        me = lax.axis_index("i")

        barrier = pltpu.get_barrier_semaphore()
        for k in range(LOG2):
            peer = me ^ (1 << k)
            pl.semaphore_signal(
                barrier, inc=1,
                device_id=(peer,), device_id_type=pl.DeviceIdType.MESH,
            )
        pl.semaphore_wait(barrier, LOG2)

        xcur = x_ref[...]
        weights = [
            (win0_ref, wout0_ref),
            (win1_ref, wout1_ref),
            (win2_ref, wout2_ref),
        ]
        for layer, (win_ref, wout_ref) in enumerate(weights):
            h = jnp.maximum(
                jnp.dot(xcur, win_ref[...], preferred_element_type=jnp.float32),
                0.0,
            )
            accum_ref[...] = jnp.dot(
                h, wout_ref[...], preferred_element_type=jnp.float32
            )

            for k in range(LOG2):
                peer = me ^ (1 << k)
                slot = layer * LOG2 + k
                rdma = pltpu.make_async_remote_copy(
                    src_ref=accum_ref,
                    dst_ref=recv_ref.at[slot],
                    send_sem=send_sems.at[slot],
                    recv_sem=recv_sems.at[slot],
                    device_id=(peer,),
                    device_id_type=pl.DeviceIdType.MESH,
                )
                rdma.start()
                rdma.wait()
                accum_ref[...] = accum_ref[...] + recv_ref[slot]

            xcur = accum_ref[...]

        out_ref[...] = xcur

    n_slots = N_LAYERS * LOG2
    return pl.pallas_call(
        body,
        out_shape=jax.ShapeDtypeStruct((b, d), jnp.float32),
        in_specs=[pl.BlockSpec(memory_space=pltpu.VMEM)] * 7,
        out_specs=pl.BlockSpec(memory_space=pltpu.VMEM),
        scratch_shapes=[
            pltpu.VMEM((b, d), jnp.float32),
            pltpu.VMEM((n_slots, b, d), jnp.float32),
            pltpu.SemaphoreType.DMA((n_slots,)),
            pltpu.SemaphoreType.DMA((n_slots,)),
        ],
        compiler_params=pltpu.CompilerParams(collective_id=0),
    )(x, Win0, Wout0, Win1, Wout1, Win2, Wout2)


# device time: 105986 ns/iter; 1.5978x vs baseline; 1.5978x over previous
import jax
import jax.numpy as jnp
from jax import lax
from jax.experimental import pallas as pl
from jax.experimental.pallas import tpu as pltpu

N_DEV = 32
LOG2 = 5
N_LAYERS = 3

RS_MASKS = (1, 2, 4, 8, 16)
AG_MASKS = tuple(reversed(RS_MASKS))


def kernel(x, Win0, Wout0, Win1, Wout1, Win2, Wout2):
    b, d = x.shape

    halves = [b >> (i + 1) for i in range(LOG2)]
    bases = [sum(halves[:i]) for i in range(LOG2)]

    def body(x_ref, win0_ref, wout0_ref, win1_ref, wout1_ref, win2_ref,
             wout2_ref, out_ref, accum_ref, recv_ref,
             rs_send_sems, rs_recv_sems, ag_send_sems, ag_recv_sems):
        me = lax.axis_index("i")

        barrier = pltpu.get_barrier_semaphore()
        for m in RS_MASKS:
            pl.semaphore_signal(
                barrier, inc=1,
                device_id=(me ^ m,), device_id_type=pl.DeviceIdType.MESH,
            )
        pl.semaphore_wait(barrier, LOG2)

        xcur = x_ref[...]
        weights = [
            (win0_ref, wout0_ref),
            (win1_ref, wout1_ref),
            (win2_ref, wout2_ref),
        ]
        for layer, (win_ref, wout_ref) in enumerate(weights):
            h = jnp.maximum(
                jnp.dot(xcur, win_ref[...], preferred_element_type=jnp.float32),
                0.0,
            )
            accum_ref[...] = jnp.dot(
                h, wout_ref[...], preferred_element_type=jnp.float32
            )

            off = me * 0
            size = b
            for i, m in enumerate(RS_MASKS):
                half = size // 2
                peer = me ^ m
                bit = (me // m) % 2
                send_off = off + half * (1 - bit)
                keep_off = off + half * bit
                rdma = pltpu.make_async_remote_copy(
                    src_ref=accum_ref.at[pl.ds(send_off, half), :],
                    dst_ref=recv_ref.at[layer, pl.ds(bases[i], half), :],
                    send_sem=rs_send_sems.at[layer, i],
                    recv_sem=rs_recv_sems.at[layer, i],
                    device_id=(peer,),
                    device_id_type=pl.DeviceIdType.MESH,
                )
                rdma.start()
                rdma.wait()
                accum_ref[pl.ds(keep_off, half), :] = (
                    accum_ref[pl.ds(keep_off, half), :]
                    + recv_ref[layer, pl.ds(bases[i], half), :]
                )
                off = keep_off
                size = half

            for j, m in enumerate(AG_MASKS):
                peer = me ^ m
                bit = (me // m) % 2
                rdma = pltpu.make_async_remote_copy(
                    src_ref=accum_ref.at[pl.ds(off, size), :],
                    dst_ref=accum_ref.at[pl.ds(off, size), :],
                    send_sem=ag_send_sems.at[layer, j],
                    recv_sem=ag_recv_sems.at[layer, j],
                    device_id=(peer,),
                    device_id_type=pl.DeviceIdType.MESH,
                )
                rdma.start()
                rdma.wait()
                off = off - size * bit
                size = size * 2

            xcur = accum_ref[...]

        out_ref[...] = xcur

    rs_rows = sum(halves)
    return pl.pallas_call(
        body,
        out_shape=jax.ShapeDtypeStruct((b, d), jnp.float32),
        in_specs=[pl.BlockSpec(memory_space=pltpu.VMEM)] * 7,
        out_specs=pl.BlockSpec(memory_space=pltpu.VMEM),
        scratch_shapes=[
            pltpu.VMEM((b, d), jnp.float32),
            pltpu.VMEM((N_LAYERS, rs_rows, d), jnp.float32),
            pltpu.SemaphoreType.DMA((N_LAYERS, LOG2)),
            pltpu.SemaphoreType.DMA((N_LAYERS, LOG2)),
            pltpu.SemaphoreType.DMA((N_LAYERS, LOG2)),
            pltpu.SemaphoreType.DMA((N_LAYERS, LOG2)),
        ],
        compiler_params=pltpu.CompilerParams(collective_id=0),
    )(x, Win0, Wout0, Win1, Wout1, Win2, Wout2)


# device time: 77197 ns/iter; 2.1937x vs baseline; 1.3729x over previous
import jax
import jax.numpy as jnp
from jax import lax
from jax.experimental import pallas as pl
from jax.experimental.pallas import tpu as pltpu

N_DEV = 32
PLANE = 8
N_PLANES = 4
N_LAYERS = 3


def kernel(x, Win0, Wout0, Win1, Wout1, Win2, Wout2):
    b, d = x.shape
    blk = b // PLANE
    sub = blk // N_PLANES

    def body(x_ref, win0_ref, wout0_ref, win1_ref, wout1_ref, win2_ref,
             wout2_ref, out_ref, accum_ref, s1_ref, s2_ref,
             rs1_send, rs1_recv, rs2_send, rs2_recv,
             ag1_send, ag1_recv, ag2_send, ag2_recv):
        me = lax.axis_index("i")
        plane = me // PLANE
        idx = me % PLANE

        def dev(p, q):
            return (p * PLANE + q,)

        barrier = pltpu.get_barrier_semaphore()
        for t in range(1, PLANE):
            pl.semaphore_signal(
                barrier, inc=1,
                device_id=dev(plane, (idx + t) % PLANE),
                device_id_type=pl.DeviceIdType.MESH,
            )
        for u in range(1, N_PLANES):
            pl.semaphore_signal(
                barrier, inc=1,
                device_id=dev((plane + u) % N_PLANES, idx),
                device_id_type=pl.DeviceIdType.MESH,
            )
        pl.semaphore_wait(barrier, (PLANE - 1) + (N_PLANES - 1))

        def recv_wait(dst_ref, recv_sem):
            pltpu.make_async_remote_copy(
                src_ref=dst_ref, dst_ref=dst_ref,
                send_sem=recv_sem, recv_sem=recv_sem,
                device_id=(me,), device_id_type=pl.DeviceIdType.MESH,
            ).wait_recv()

        xcur = x_ref[...]
        weights = [
            (win0_ref, wout0_ref),
            (win1_ref, wout1_ref),
            (win2_ref, wout2_ref),
        ]
        for layer, (win_ref, wout_ref) in enumerate(weights):
            h = jnp.maximum(
                jnp.dot(xcur, win_ref[...], preferred_element_type=jnp.float32),
                0.0,
            )
            accum_ref[...] = jnp.dot(
                h, wout_ref[...], preferred_element_type=jnp.float32
            )

            my_blk = idx * blk
            my_sub = my_blk + plane * sub

            sends = []
            for t in range(1, PLANE):
                q = (idx + t) % PLANE
                rdma = pltpu.make_async_remote_copy(
                    src_ref=accum_ref.at[pl.ds(q * blk, blk), :],
                    dst_ref=s1_ref.at[layer, t - 1],
                    send_sem=rs1_send.at[layer, t - 1],
                    recv_sem=rs1_recv.at[layer, t - 1],
                    device_id=dev(plane, q),
                    device_id_type=pl.DeviceIdType.MESH,
                )
                rdma.start()
                sends.append(rdma)
            for t in range(1, PLANE):
                recv_wait(s1_ref.at[layer, t - 1], rs1_recv.at[layer, t - 1])
                accum_ref[pl.ds(my_blk, blk), :] = (
                    accum_ref[pl.ds(my_blk, blk), :] + s1_ref[layer, t - 1]
                )
            for rdma in sends:
                rdma.wait_send()

            sends = []
            for u in range(1, N_PLANES):
                p = (plane + u) % N_PLANES
                rdma = pltpu.make_async_remote_copy(
                    src_ref=accum_ref.at[pl.ds(my_blk + p * sub, sub), :],
                    dst_ref=s2_ref.at[layer, u - 1],
                    send_sem=rs2_send.at[layer, u - 1],
                    recv_sem=rs2_recv.at[layer, u - 1],
                    device_id=dev(p, idx),
                    device_id_type=pl.DeviceIdType.MESH,
                )
                rdma.start()
                sends.append(rdma)
            for u in range(1, N_PLANES):
                recv_wait(s2_ref.at[layer, u - 1], rs2_recv.at[layer, u - 1])
                accum_ref[pl.ds(my_sub, sub), :] = (
                    accum_ref[pl.ds(my_sub, sub), :] + s2_ref[layer, u - 1]
                )
            for rdma in sends:
                rdma.wait_send()

            sends = []
            for u in range(1, N_PLANES):
                p = (plane + u) % N_PLANES
                rdma = pltpu.make_async_remote_copy(
                    src_ref=accum_ref.at[pl.ds(my_sub, sub), :],
                    dst_ref=accum_ref.at[pl.ds(my_sub, sub), :],
                    send_sem=ag1_send.at[layer, u - 1],
                    recv_sem=ag1_recv.at[layer, u - 1],
                    device_id=dev(p, idx),
                    device_id_type=pl.DeviceIdType.MESH,
                )
                rdma.start()
                sends.append(rdma)
            for u in range(1, N_PLANES):
                recv_wait(
                    accum_ref.at[pl.ds(my_blk, sub), :],
                    ag1_recv.at[layer, u - 1],
                )
            for rdma in sends:
                rdma.wait_send()

            sends = []
            for t in range(1, PLANE):
                q = (idx + t) % PLANE
                rdma = pltpu.make_async_remote_copy(
                    src_ref=accum_ref.at[pl.ds(my_blk, blk), :],
                    dst_ref=accum_ref.at[pl.ds(my_blk, blk), :],
                    send_sem=ag2_send.at[layer, t - 1],
                    recv_sem=ag2_recv.at[layer, t - 1],
                    device_id=dev(plane, q),
                    device_id_type=pl.DeviceIdType.MESH,
                )
                rdma.start()
                sends.append(rdma)
            for t in range(1, PLANE):
                recv_wait(
                    accum_ref.at[pl.ds(my_blk, blk), :],
                    ag2_recv.at[layer, t - 1],
                )
            for rdma in sends:
                rdma.wait_send()

            xcur = accum_ref[...]

        out_ref[...] = xcur

    return pl.pallas_call(
        body,
        out_shape=jax.ShapeDtypeStruct((b, d), jnp.float32),
        in_specs=[pl.BlockSpec(memory_space=pltpu.VMEM)] * 7,
        out_specs=pl.BlockSpec(memory_space=pltpu.VMEM),
        scratch_shapes=[
            pltpu.VMEM((b, d), jnp.float32),
            pltpu.VMEM((N_LAYERS, PLANE - 1, blk, d), jnp.float32),
            pltpu.VMEM((N_LAYERS, N_PLANES - 1, sub, d), jnp.float32),
            pltpu.SemaphoreType.DMA((N_LAYERS, PLANE - 1)),
            pltpu.SemaphoreType.DMA((N_LAYERS, PLANE - 1)),
            pltpu.SemaphoreType.DMA((N_LAYERS, N_PLANES - 1)),
            pltpu.SemaphoreType.DMA((N_LAYERS, N_PLANES - 1)),
            pltpu.SemaphoreType.DMA((N_LAYERS, N_PLANES - 1)),
            pltpu.SemaphoreType.DMA((N_LAYERS, N_PLANES - 1)),
            pltpu.SemaphoreType.DMA((N_LAYERS, PLANE - 1)),
            pltpu.SemaphoreType.DMA((N_LAYERS, PLANE - 1)),
        ],
        compiler_params=pltpu.CompilerParams(collective_id=0),
    )(x, Win0, Wout0, Win1, Wout1, Win2, Wout2)


# device time: 70069 ns/iter; 2.4169x vs baseline; 1.1017x over previous
import jax
import jax.numpy as jnp
from jax import lax
from jax.experimental import pallas as pl
from jax.experimental.pallas import tpu as pltpu

N_DEV = 32
N_LAYERS = 3


def kernel(x, Win0, Wout0, Win1, Wout1, Win2, Wout2):
    b, d = x.shape
    sub = b // N_DEV

    def body(x_ref, win0_ref, wout0_ref, win1_ref, wout1_ref, win2_ref,
             wout2_ref, out_ref, accum_ref, stage_ref,
             rs_send, rs_recv, ag_send, ag_recv):
        me = lax.axis_index("i")

        barrier = pltpu.get_barrier_semaphore()
        for t in range(1, N_DEV):
            pl.semaphore_signal(
                barrier, inc=1,
                device_id=((me + t) % N_DEV,),
                device_id_type=pl.DeviceIdType.MESH,
            )
        pl.semaphore_wait(barrier, N_DEV - 1)

        def recv_wait(dst_ref, recv_sem):
            pltpu.make_async_remote_copy(
                src_ref=dst_ref, dst_ref=dst_ref,
                send_sem=recv_sem, recv_sem=recv_sem,
                device_id=(me,), device_id_type=pl.DeviceIdType.MESH,
            ).wait_recv()

        xcur = x_ref[...]
        weights = [
            (win0_ref, wout0_ref),
            (win1_ref, wout1_ref),
            (win2_ref, wout2_ref),
        ]
        for layer, (win_ref, wout_ref) in enumerate(weights):
            h = jnp.maximum(
                jnp.dot(xcur, win_ref[...], preferred_element_type=jnp.float32),
                0.0,
            )
            accum_ref[...] = jnp.dot(
                h, wout_ref[...], preferred_element_type=jnp.float32
            )

            my_off = me * sub

            sends = []
            for t in range(1, N_DEV):
                dst = (me + t) % N_DEV
                rdma = pltpu.make_async_remote_copy(
                    src_ref=accum_ref.at[pl.ds(dst * sub, sub), :],
                    dst_ref=stage_ref.at[layer, t - 1],
                    send_sem=rs_send.at[layer, t - 1],
                    recv_sem=rs_recv.at[layer, t - 1],
                    device_id=(dst,),
                    device_id_type=pl.DeviceIdType.MESH,
                )
                rdma.start()
                sends.append(rdma)
            for t in range(1, N_DEV):
                recv_wait(stage_ref.at[layer, t - 1], rs_recv.at[layer, t - 1])
            accum_ref[pl.ds(my_off, sub), :] = (
                accum_ref[pl.ds(my_off, sub), :]
                + jnp.sum(stage_ref[layer], axis=0)
            )
            for rdma in sends:
                rdma.wait_send()

            sends = []
            for t in range(1, N_DEV):
                dst = (me + t) % N_DEV
                rdma = pltpu.make_async_remote_copy(
                    src_ref=accum_ref.at[pl.ds(my_off, sub), :],
                    dst_ref=accum_ref.at[pl.ds(my_off, sub), :],
                    send_sem=ag_send.at[layer, t - 1],
                    recv_sem=ag_recv.at[layer, t - 1],
                    device_id=(dst,),
                    device_id_type=pl.DeviceIdType.MESH,
                )
                rdma.start()
                sends.append(rdma)
            for t in range(1, N_DEV):
                recv_wait(
                    accum_ref.at[pl.ds(my_off, sub), :],
                    ag_recv.at[layer, t - 1],
                )
            for rdma in sends:
                rdma.wait_send()

            xcur = accum_ref[...]

        out_ref[...] = xcur

    return pl.pallas_call(
        body,
        out_shape=jax.ShapeDtypeStruct((b, d), jnp.float32),
        in_specs=[pl.BlockSpec(memory_space=pltpu.VMEM)] * 7,
        out_specs=pl.BlockSpec(memory_space=pltpu.VMEM),
        scratch_shapes=[
            pltpu.VMEM((b, d), jnp.float32),
            pltpu.VMEM((N_LAYERS, N_DEV - 1, sub, d), jnp.float32),
            pltpu.SemaphoreType.DMA((N_LAYERS, N_DEV - 1)),
            pltpu.SemaphoreType.DMA((N_LAYERS, N_DEV - 1)),
            pltpu.SemaphoreType.DMA((N_LAYERS, N_DEV - 1)),
            pltpu.SemaphoreType.DMA((N_LAYERS, N_DEV - 1)),
        ],
        compiler_params=pltpu.CompilerParams(collective_id=0),
    )(x, Win0, Wout0, Win1, Wout1, Win2, Wout2)


# device time: 59224 ns/iter; 2.8594x vs baseline; 1.1831x over previous
import jax
import jax.numpy as jnp
from jax import lax
from jax.experimental import pallas as pl
from jax.experimental.pallas import tpu as pltpu

N_DEV = 32
N_LAYERS = 3
N_HALF = 2


def kernel(x, Win0, Wout0, Win1, Wout1, Win2, Wout2):
    b, d = x.shape
    hb = b // N_HALF
    sub = hb // N_DEV

    def body(x_ref, win0_ref, wout0_ref, win1_ref, wout1_ref, win2_ref,
             wout2_ref, out_ref, accum_ref, stage_ref,
             rs_send, rs_recv, ag_send, ag_recv):
        me = lax.axis_index("i")

        barrier = pltpu.get_barrier_semaphore()
        for t in range(1, N_DEV):
            pl.semaphore_signal(
                barrier, inc=1,
                device_id=((me + t) % N_DEV,),
                device_id_type=pl.DeviceIdType.MESH,
            )
        pl.semaphore_wait(barrier, N_DEV - 1)

        def recv_wait(dst_ref, recv_sem):
            pltpu.make_async_remote_copy(
                src_ref=dst_ref, dst_ref=dst_ref,
                send_sem=recv_sem, recv_sem=recv_sem,
                device_id=(me,), device_id_type=pl.DeviceIdType.MESH,
            ).wait_recv()

        weights = [
            (win0_ref, wout0_ref),
            (win1_ref, wout1_ref),
            (win2_ref, wout2_ref),
        ]
        prev_rs = [[], []]
        prev_ag = [[], []]

        for layer, (win_ref, wout_ref) in enumerate(weights):
            cur_rs = [[], []]
            for half in range(N_HALF):
                row0 = half * hb
                my_off = row0 + me * sub
                if layer > 0:
                    for t in range(1, N_DEV):
                        recv_wait(
                            accum_ref.at[pl.ds(my_off, sub), :],
                            ag_recv.at[half, t - 1],
                        )
                    xh = accum_ref[pl.ds(row0, hb), :]
                else:
                    xh = x_ref[pl.ds(row0, hb), :]
                for rdma in prev_rs[half]:
                    rdma.wait_send()
                for rdma in prev_ag[half]:
                    rdma.wait_send()

                h = jnp.maximum(
                    jnp.dot(xh, win_ref[...],
                            preferred_element_type=jnp.float32),
                    0.0,
                )
                accum_ref[pl.ds(row0, hb), :] = jnp.dot(
                    h, wout_ref[...], preferred_element_type=jnp.float32
                )

                for t in range(1, N_DEV):
                    dst = (me + t) % N_DEV
                    rdma = pltpu.make_async_remote_copy(
                        src_ref=accum_ref.at[pl.ds(row0 + dst * sub, sub), :],
                        dst_ref=stage_ref.at[half, t - 1],
                        send_sem=rs_send.at[half, t - 1],
                        recv_sem=rs_recv.at[half, t - 1],
                        device_id=(dst,),
                        device_id_type=pl.DeviceIdType.MESH,
                    )
                    rdma.start()
                    cur_rs[half].append(rdma)

            cur_ag = [[], []]
            for half in range(N_HALF):
                row0 = half * hb
                my_off = row0 + me * sub
                for t in range(1, N_DEV):
                    recv_wait(stage_ref.at[half, t - 1],
                              rs_recv.at[half, t - 1])
                accum_ref[pl.ds(my_off, sub), :] = (
                    accum_ref[pl.ds(my_off, sub), :]
                    + jnp.sum(stage_ref[half], axis=0)
                )
                for t in range(1, N_DEV):
                    dst = (me + t) % N_DEV
                    rdma = pltpu.make_async_remote_copy(
                        src_ref=accum_ref.at[pl.ds(my_off, sub), :],
                        dst_ref=accum_ref.at[pl.ds(my_off, sub), :],
                        send_sem=ag_send.at[half, t - 1],
                        recv_sem=ag_recv.at[half, t - 1],
                        device_id=(dst,),
                        device_id_type=pl.DeviceIdType.MESH,
                    )
                    rdma.start()
                    cur_ag[half].append(rdma)

            prev_rs = cur_rs
            prev_ag = cur_ag

        for half in range(N_HALF):
            my_off = half * hb + me * sub
            for t in range(1, N_DEV):
                recv_wait(accum_ref.at[pl.ds(my_off, sub), :],
                          ag_recv.at[half, t - 1])
            for rdma in prev_rs[half]:
                rdma.wait_send()
            for rdma in prev_ag[half]:
                rdma.wait_send()
        out_ref[...] = accum_ref[...]

    return pl.pallas_call(
        body,
        out_shape=jax.ShapeDtypeStruct((b, d), jnp.float32),
        in_specs=[pl.BlockSpec(memory_space=pltpu.VMEM)] * 7,
        out_specs=pl.BlockSpec(memory_space=pltpu.VMEM),
        scratch_shapes=[
            pltpu.VMEM((b, d), jnp.float32),
            pltpu.VMEM((N_HALF, N_DEV - 1, sub, d), jnp.float32),
            pltpu.SemaphoreType.DMA((N_HALF, N_DEV - 1)),
            pltpu.SemaphoreType.DMA((N_HALF, N_DEV - 1)),
            pltpu.SemaphoreType.DMA((N_HALF, N_DEV - 1)),
            pltpu.SemaphoreType.DMA((N_HALF, N_DEV - 1)),
        ],
        compiler_params=pltpu.CompilerParams(collective_id=0),
    )(x, Win0, Wout0, Win1, Wout1, Win2, Wout2)


# device time: 59120 ns/iter; 2.8645x vs baseline; 1.0018x over previous
import jax
import jax.numpy as jnp
from jax import lax
from jax.experimental import pallas as pl
from jax.experimental.pallas import tpu as pltpu

N_DEV = 32
N_LAYERS = 3
N_HALF = 2


def kernel(x, Win0, Wout0, Win1, Wout1, Win2, Wout2):
    b, d = x.shape
    hb = b // N_HALF
    sub = hb // N_DEV

    def body(x_ref, win0_ref, wout0_ref, win1_ref, wout1_ref, win2_ref,
             wout2_ref, out_ref, stage_ref,
             rs_send, rs_recv, ag_send, ag_recv):
        me = lax.axis_index("i")

        barrier = pltpu.get_barrier_semaphore()
        for t in range(1, N_DEV):
            pl.semaphore_signal(
                barrier, inc=1,
                device_id=((me + t) % N_DEV,),
                device_id_type=pl.DeviceIdType.MESH,
            )
        pl.semaphore_wait(barrier, N_DEV - 1)

        def recv_wait(dst_ref, recv_sem):
            pltpu.make_async_remote_copy(
                src_ref=dst_ref, dst_ref=dst_ref,
                send_sem=recv_sem, recv_sem=recv_sem,
                device_id=(me,), device_id_type=pl.DeviceIdType.MESH,
            ).wait_recv()

        weights = [
            (win0_ref, wout0_ref),
            (win1_ref, wout1_ref),
            (win2_ref, wout2_ref),
        ]
        prev_rs = [[], []]
        prev_ag = [[], []]

        for layer, (win_ref, wout_ref) in enumerate(weights):
            cur_rs = [[], []]
            for half in range(N_HALF):
                row0 = half * hb
                my_off = row0 + me * sub
                if layer > 0:
                    for t in range(1, N_DEV):
                        recv_wait(
                            out_ref.at[pl.ds(my_off, sub), :],
                            ag_recv.at[half, t - 1],
                        )
                    xh = out_ref[pl.ds(row0, hb), :]
                else:
                    xh = x_ref[pl.ds(row0, hb), :]
                for rdma in prev_rs[half]:
                    rdma.wait_send()
                for rdma in prev_ag[half]:
                    rdma.wait_send()

                h = jnp.maximum(
                    jnp.dot(xh, win_ref[...],
                            preferred_element_type=jnp.float32),
                    0.0,
                )
                out_ref[pl.ds(row0, hb), :] = jnp.dot(
                    h, wout_ref[...], preferred_element_type=jnp.float32
                )

                for t in range(1, N_DEV):
                    dst = (me + t) % N_DEV
                    rdma = pltpu.make_async_remote_copy(
                        src_ref=out_ref.at[pl.ds(row0 + dst * sub, sub), :],
                        dst_ref=stage_ref.at[half, t - 1],
                        send_sem=rs_send.at[half, t - 1],
                        recv_sem=rs_recv.at[half, t - 1],
                        device_id=(dst,),
                        device_id_type=pl.DeviceIdType.MESH,
                    )
                    rdma.start()
                    cur_rs[half].append(rdma)

            cur_ag = [[], []]
            for half in range(N_HALF):
                row0 = half * hb
                my_off = row0 + me * sub
                for t in range(1, N_DEV):
                    recv_wait(stage_ref.at[half, t - 1],
                              rs_recv.at[half, t - 1])
                out_ref[pl.ds(my_off, sub), :] = (
                    out_ref[pl.ds(my_off, sub), :]
                    + jnp.sum(stage_ref[half], axis=0)
                )
                for t in range(1, N_DEV):
                    dst = (me + t) % N_DEV
                    rdma = pltpu.make_async_remote_copy(
                        src_ref=out_ref.at[pl.ds(my_off, sub), :],
                        dst_ref=out_ref.at[pl.ds(my_off, sub), :],
                        send_sem=ag_send.at[half, t - 1],
                        recv_sem=ag_recv.at[half, t - 1],
                        device_id=(dst,),
                        device_id_type=pl.DeviceIdType.MESH,
                    )
                    rdma.start()
                    cur_ag[half].append(rdma)

            prev_rs = cur_rs
            prev_ag = cur_ag

        for half in range(N_HALF):
            my_off = half * hb + me * sub
            for t in range(1, N_DEV):
                recv_wait(out_ref.at[pl.ds(my_off, sub), :],
                          ag_recv.at[half, t - 1])
            for rdma in prev_rs[half]:
                rdma.wait_send()
            for rdma in prev_ag[half]:
                rdma.wait_send()

    return pl.pallas_call(
        body,
        out_shape=jax.ShapeDtypeStruct((b, d), jnp.float32),
        in_specs=[pl.BlockSpec(memory_space=pltpu.VMEM)] * 7,
        out_specs=pl.BlockSpec(memory_space=pltpu.VMEM),
        scratch_shapes=[
            pltpu.VMEM((N_HALF, N_DEV - 1, sub, d), jnp.float32),
            pltpu.SemaphoreType.DMA((N_HALF, N_DEV - 1)),
            pltpu.SemaphoreType.DMA((N_HALF, N_DEV - 1)),
            pltpu.SemaphoreType.DMA((N_HALF, N_DEV - 1)),
            pltpu.SemaphoreType.DMA((N_HALF, N_DEV - 1)),
        ],
        compiler_params=pltpu.CompilerParams(collective_id=0),
    )(x, Win0, Wout0, Win1, Wout1, Win2, Wout2)
